# Initial kernel scaffold; baseline (speedup 1.0000x reference)
import jax
import jax.numpy as jnp
from jax.experimental import pallas as pl


def kernel(features, edge_index, W1, al1, ar1, b1, W2, al2, ar2, b2, W3, al3, ar3, b3):
    raise NotImplementedError("write your pallas kernel here")


# speedup vs baseline: 2.4923x; 2.4923x over previous
import functools

import jax
import jax.numpy as jnp
from jax import lax
from jax.experimental import pallas as pl
from jax.experimental.pallas import tpu as pltpu
from jax.experimental.pallas import tpu_sc as plsc

N_NODES = 50000
N_EDGES = 800000

NSC = 16
LANES = 16
N_PAD = 50176
ROWS_PER_SUB = N_PAD // NSC
C = 64
E_PAD = 802816
BN = 512


QW = 16


def _mm_body(pre_bias, pre_slope, nx, nq, *refs):
    x_refs = refs[:nx]
    W_ref, al_ref, ar_ref, b_ref = refs[nx:nx + 4]
    q_refs = refs[nx + 4:nx + 4 + nq]
    el_ref, er_ref = refs[nx + 4 + nq:]
    if nx == 1:
        x = x_refs[0][...]
    else:
        x = jnp.concatenate([r[...] for r in x_refs], axis=1)
    if pre_bias:
        x = x + b_ref[...]
        x = jnp.where(x > 0, x, pre_slope * x)
    h = jnp.dot(x, W_ref[...], preferred_element_type=jnp.float32)
    pad = jnp.zeros((h.shape[0], 128 - QW), jnp.float32)
    for q in range(nq):
        q_refs[q][...] = jnp.concatenate(
            [h[:, q * QW:(q + 1) * QW], pad], axis=1)
    el_ref[...] = jnp.sum(h * al_ref[...][None, :], axis=1)
    er_ref[...] = jnp.sum(h * ar_ref[...][None, :], axis=1)


def _mm_stage(xs, W, al, ar, b_prev, pre_slope):
    fin = sum(x.shape[1] for x in xs)
    fout = W.shape[1]
    nq = fout // QW
    in_specs = [pl.BlockSpec((BN, x.shape[1]), lambda i: (i, 0)) for x in xs]
    args = list(xs)
    in_specs += [
        pl.BlockSpec((fin, fout), lambda i: (0, 0)),
        pl.BlockSpec((fout,), lambda i: (0,)),
        pl.BlockSpec((fout,), lambda i: (0,)),
        pl.BlockSpec((fin,), lambda i: (0,)),
    ]
    args += [W, al, ar,
             jnp.zeros((fin,), jnp.float32) if b_prev is None else b_prev]
    body = functools.partial(_mm_body, b_prev is not None, pre_slope,
                             len(xs), nq)
    out = pl.pallas_call(
        body,
        grid=(N_PAD // BN,),
        in_specs=in_specs,
        out_specs=(
            [pl.BlockSpec((BN, 128), lambda i: (i, 0)) for _ in range(nq)]
            + [pl.BlockSpec((BN,), lambda i: (i,))] * 2),
        out_shape=(
            [jax.ShapeDtypeStruct((N_PAD, 128), jnp.float32)
             for _ in range(nq)]
            + [jax.ShapeDtypeStruct((N_PAD,), jnp.float32)] * 2),
    )(*args)
    return out[:nq], out[nq], out[nq + 1]


_MESH = plsc.VectorSubcoreMesh(core_axis_name="c", subcore_axis_name="s")

_CHUNKS_P1 = E_PAD // (32 * C)


def _edge1_body(src_hbm, dst_hbm, el_hbm, er_hbm,
                ex_hbm, p0_hbm, p1_hbm,
                srcv, dstv, elg, erg, exv, zv, denom_sp, sem):
    cid = lax.axis_index("c")
    sid = lax.axis_index("s")
    wid = sid * 2 + cid

    def zb(i, c2):
        zv[pl.ds(i * LANES, LANES)] = jnp.zeros((LANES,), jnp.float32)
        return c2
    lax.fori_loop(0, ROWS_PER_SUB // LANES, zb, 0)
    pltpu.sync_copy(zv, denom_sp.at[pl.ds(sid * ROWS_PER_SUB, ROWS_PER_SUB)])
    plsc.subcore_barrier()

    def chunk(j, carry):
        off = (wid * _CHUNKS_P1 + j) * C
        pltpu.sync_copy(src_hbm.at[pl.ds(off, C)], srcv)
        pltpu.sync_copy(dst_hbm.at[pl.ds(off, C)], dstv)
        pltpu.async_copy(el_hbm.at[srcv], elg, sem).wait()
        pltpu.async_copy(er_hbm.at[dstv], erg, sem).wait()

        def vec(i, c2):
            s = elg[pl.ds(i * LANES, LANES)] + erg[pl.ds(i * LANES, LANES)]
            e = jnp.where(s > 0, s, 0.2 * s)
            exv[pl.ds(i * LANES, LANES)] = jnp.exp(e)
            return c2
        lax.fori_loop(0, C // LANES, vec, 0)
        pltpu.sync_copy(exv, ex_hbm.at[pl.ds(off, C)])
        pltpu.sync_copy(exv, denom_sp.at[dstv], add=True)
        return carry

    lax.fori_loop(0, _CHUNKS_P1, chunk, 0)
    plsc.subcore_barrier()
    rs = pl.ds(sid * ROWS_PER_SUB, ROWS_PER_SUB)
    pltpu.sync_copy(denom_sp.at[rs], zv)

    @pl.when(cid == 0)
    def _():
        pltpu.sync_copy(zv, p0_hbm.at[rs])

    @pl.when(cid == 1)
    def _():
        pltpu.sync_copy(zv, p1_hbm.at[rs])


_edge1 = functools.partial(
    pl.kernel, _edge1_body, mesh=_MESH,
    out_type=[
        jax.ShapeDtypeStruct((E_PAD,), jnp.float32),
        jax.ShapeDtypeStruct((N_PAD,), jnp.float32),
        jax.ShapeDtypeStruct((N_PAD,), jnp.float32),
    ],
    scratch_types=[
        pltpu.VMEM((C,), jnp.int32),
        pltpu.VMEM((C,), jnp.int32),
        pltpu.VMEM((C,), jnp.float32),
        pltpu.VMEM((C,), jnp.float32),
        pltpu.VMEM((C,), jnp.float32),
        pltpu.VMEM((ROWS_PER_SUB,), jnp.float32),
        pltpu.VMEM_SHARED((N_PAD,), jnp.float32),
        pltpu.SemaphoreType.DMA,
    ],
)()


_CHUNKS_P2 = E_PAD // (NSC * C)


_ST_ROWS = 392
_ST_CH = ROWS_PER_SUB // _ST_ROWS
N_HALF = N_PAD // 2
_HROWS = N_HALF // NSC


_WB = QW * ROWS_PER_SUB // 8


def _edge2_body(both_cores,
                src_hbm, dst_hbm, ex_hbm, p0_hbm, p1_hbm, ha_hbm, hb_hbm,
                oa_hbm, ob_hbm,
                srcv, dstv, exv, d0, d1, alphav, rows, rows16, valv, idxv,
                st1, acc_sp, sem):
    cid = lax.axis_index("c")
    sid = lax.axis_index("s")

    def run(h_hbm, o_hbm):
        def zb(r, c2):
            st1[pl.ds(r * LANES, LANES)] = jnp.zeros((LANES,), jnp.float32)
            return c2
        lax.fori_loop(0, _WB // LANES, zb, 0)

        def zcp(k, c2):
            pltpu.sync_copy(
                st1, acc_sp.at[pl.ds(sid * QW * ROWS_PER_SUB + k * _WB, _WB)])
            return c2
        lax.fori_loop(0, 8, zcp, 0)
        plsc.subcore_barrier()

        def chunk(j, carry):
            off = (sid * _CHUNKS_P2 + j) * C
            pltpu.sync_copy(src_hbm.at[pl.ds(off, C)], srcv)
            pltpu.sync_copy(dst_hbm.at[pl.ds(off, C)], dstv)
            pltpu.sync_copy(ex_hbm.at[pl.ds(off, C)], exv)
            pltpu.async_copy(p0_hbm.at[dstv], d0, sem).wait()
            pltpu.async_copy(p1_hbm.at[dstv], d1, sem).wait()

            def vec(i, c2):
                sl = pl.ds(i * LANES, LANES)
                den = d0[sl] + d1[sl]
                alphav[sl] = exv[sl] / (den + 1e-16)
                return c2
            lax.fori_loop(0, C // LANES, vec, 0)

            pltpu.async_copy(h_hbm.at[srcv], rows, sem).wait()

            iota = lax.iota(jnp.int32, LANES)

            def scale(g, c2):
                a16 = alphav[pl.ds(g * LANES, LANES)]
                d16 = dstv[pl.ds(g * LANES, LANES)]
                for k in range(LANES):
                    i = g * LANES + k
                    a = a16[k]
                    d = d16[k]
                    rows16[pl.ds(i * QW, QW)] = a * rows[i, pl.ds(0, LANES)]
                    pltpu.sync_copy(rows16.at[pl.ds(i * QW, QW)],
                                    acc_sp.at[d * QW + iota], add=True)
                return c2
            lax.fori_loop(0, C // LANES, scale, 0)
            return carry

        lax.fori_loop(0, _CHUNKS_P2, chunk, 0)
        plsc.subcore_barrier()

        def wcp(k, c2):
            rs = pl.ds(sid * QW * ROWS_PER_SUB + k * _WB, _WB)
            pltpu.sync_copy(acc_sp.at[rs], st1)
            pltpu.sync_copy(st1, o_hbm.at[rs])
            return c2
        lax.fori_loop(0, 8, wcp, 0)
        plsc.subcore_barrier()

    @pl.when(cid == 0)
    def _():
        run(ha_hbm, oa_hbm)

    if both_cores:
        @pl.when(cid == 1)
        def _():
            run(hb_hbm, ob_hbm)


def _edge2(both_cores, src, dst, ex, p0, p1, ha, hb):
    body = functools.partial(_edge2_body, both_cores)
    outs = [jax.ShapeDtypeStruct((N_PAD * QW,), jnp.float32),
            jax.ShapeDtypeStruct((N_PAD * QW,), jnp.float32)]
    k = functools.partial(
        pl.kernel, body, mesh=_MESH,
        out_type=outs,
        scratch_types=[
            pltpu.VMEM((C,), jnp.int32),
            pltpu.VMEM((C,), jnp.int32),
            pltpu.VMEM((C,), jnp.float32),
            pltpu.VMEM((C,), jnp.float32),
            pltpu.VMEM((C,), jnp.float32),
            pltpu.VMEM((C,), jnp.float32),
            pltpu.VMEM((C, 128), jnp.float32),
            pltpu.VMEM((C * QW,), jnp.float32),
            pltpu.VMEM((C,), jnp.float32),
            pltpu.VMEM((C,), jnp.int32),
            pltpu.VMEM((_WB,), jnp.float32),
            pltpu.VMEM_SHARED((N_PAD * QW,), jnp.float32),
            pltpu.SemaphoreType.DMA,
        ],
    )()
    oa, ob = k(src, dst, ex, p0, p1, ha, hb)
    return oa.reshape(N_PAD, QW), ob.reshape(N_PAD, QW)


def _epi_body(oa_ref, b_ref, out_ref):
    out_ref[...] = oa_ref[...] + b_ref[...]


def _epilogue(oa, b_pad):
    w = oa.shape[1]
    return pl.pallas_call(
        _epi_body,
        grid=(N_PAD // BN,),
        in_specs=[
            pl.BlockSpec((BN, w), lambda i: (i, 0)),
            pl.BlockSpec((w,), lambda i: (0,)),
        ],
        out_specs=pl.BlockSpec((BN, w), lambda i: (i, 0)),
        out_shape=jax.ShapeDtypeStruct((N_PAD, w), jnp.float32),
    )(oa, b_pad)


def _gat_layer(xs, src, dst, W, al, ar, b_prev):
    qs, el, er = _mm_stage(xs, W, al, ar, b_prev, 0.01)
    ex, p0, p1 = _edge1(src, dst, el, er)
    outs = []
    if len(qs) == 1:
        o, _ = _edge2(True, src, dst, ex, p0, p1, qs[0], qs[0])
        outs.append(o)
    else:
        for i in range(0, len(qs), 2):
            oa, ob = _edge2(True, src, dst, ex, p0, p1, qs[i], qs[i + 1])
            outs += [oa, ob]
    return outs


@jax.jit
def _run(features, edge_index,
         W1, al1, ar1, b1, W2, al2, ar2, b2, W3, al3, ar3, b3):
    src = jnp.full((E_PAD,), N_NODES, jnp.int32).at[:N_EDGES].set(
        edge_index[0].astype(jnp.int32))
    dst = jnp.full((E_PAD,), N_NODES, jnp.int32).at[:N_EDGES].set(
        edge_index[1].astype(jnp.int32))
    x0 = jnp.zeros((N_PAD, 3), jnp.float32).at[:N_NODES].set(features)

    W3p = jnp.zeros((64, 16), jnp.float32).at[:, :3].set(W3)
    al3p = jnp.zeros((16,), jnp.float32).at[:3].set(al3)
    ar3p = jnp.zeros((16,), jnp.float32).at[:3].set(ar3)
    b3p = jnp.zeros((16,), jnp.float32).at[:3].set(b3)

    xs = _gat_layer([x0], src, dst, W1, al1, ar1, None)
    xs = _gat_layer(xs, src, dst, W2, al2, ar2, b1)
    xs = _gat_layer(xs, src, dst, W3p, al3p, ar3p, b2)
    out = _epilogue(xs[0], b3p)
    return out[:N_NODES, :3]


def kernel(features, edge_index, W1, al1, ar1, b1, W2, al2, ar2, b2,
           W3, al3, ar3, b3):
    return _run(features, edge_index, W1, al1, ar1, b1,
                W2, al2, ar2, b2, W3, al3, ar3, b3)
